# baseline (speedup 1.0000x reference)
import functools

import jax
import jax.numpy as jnp
from jax import lax
from jax.experimental import pallas as pl
from jax.experimental.pallas import tpu as pltpu
from jax.experimental.pallas import tpu_sc as plsc

N = 10000
E = 320000
D = 128

NC = 2
NS = 16
NW = NC * NS
EPW = E // NW
C = 80
NCHUNK = EPW // C
NCHT = E // C
SB = 2
NIT = 31
PTAIL = NCHUNK - NIT * 2 * SB
NPD = 10240
RPT = NPD // NS
YRT = N // NS

_mesh = plsc.VectorSubcoreMesh(core_axis_name="c", subcore_axis_name="s",
                               num_cores=NC, num_subcores=NS)


def _deg_body(edges_hbm, zeros_hbm, ones_hbm, deg_out, dst_v, ones_v, acc_sh,
              sem_i, sem_s):
    c = lax.axis_index("c")
    s = lax.axis_index("s")
    wid = s * NC + c
    pltpu.sync_copy(zeros_hbm.at[pl.ds(s * RPT, RPT)], acc_sh.at[pl.ds(s * RPT, RPT)])
    pltpu.sync_copy(ones_hbm, ones_v)
    plsc.subcore_barrier()
    base = wid * NCHUNK

    def fire_idx(setn, cstart):
        for b in range(SB):
            pltpu.async_copy(edges_hbm.at[base + cstart + b, 1],
                             dst_v.at[setn, b], sem_i)

    def drain_idx(setn):
        for b in range(SB):
            pltpu.make_async_copy(edges_hbm.at[0, 1], dst_v.at[setn, b],
                                  sem_i).wait()

    def fire_scat(setn):
        for b in range(SB):
            pltpu.async_copy(ones_v, acc_sh.at[dst_v.at[setn, b]], sem_s,
                             add=True)

    def drain_scat(setn):
        for b in range(SB):
            pltpu.make_async_copy(ones_v, acc_sh.at[dst_v.at[setn, b]],
                                  sem_s).wait()

    fire_idx(0, 0)

    def body(k, carry):
        drain_idx(0)
        fire_scat(0)

        @pl.when(k > 0)
        def _():
            drain_scat(1)

        fire_idx(1, k * 2 * SB + SB)
        drain_idx(1)
        fire_scat(1)
        drain_scat(0)

        @pl.when(k < NIT - 1)
        def _():
            fire_idx(0, (k + 1) * 2 * SB)

        return carry

    lax.fori_loop(0, NIT, body, 0)
    drain_scat(1)
    for t in range(PTAIL):
        pltpu.sync_copy(edges_hbm.at[base + NIT * 2 * SB + t, 1], dst_v.at[0, t])
        pltpu.sync_copy(ones_v, acc_sh.at[dst_v.at[0, t]], add=True)
    plsc.subcore_barrier()
    pltpu.sync_copy(acc_sh.at[pl.ds(s * RPT, RPT)],
                    deg_out.at[c, pl.ds(s * RPT, RPT)])


_deg_kernel = functools.partial(
    pl.kernel,
    out_type=jax.ShapeDtypeStruct((NC, NPD), jnp.float32),
    mesh=_mesh,
    scratch_types=[
        pltpu.VMEM((2, SB, C), jnp.int32),
        pltpu.VMEM((C,), jnp.float32),
        pltpu.VMEM_SHARED((NPD,), jnp.float32),
        pltpu.SemaphoreType.DMA,
        pltpu.SemaphoreType.DMA,
    ],
)(_deg_body)


def _prop_body(edges_hbm, y_hbm, zeros_hbm, acc_out,
               idx2, rows, acc_sh, sem_i, sem_g, sem_s):
    c = lax.axis_index("c")
    s = lax.axis_index("s")
    wid = s * NC + c

    @pl.when(c == 0)
    def _():
        @pl.when(s < NS - 1)
        def _():
            pltpu.sync_copy(y_hbm.at[pl.ds(s * RPT, RPT)],
                            acc_sh.at[pl.ds(s * RPT, RPT)])

        @pl.when(s == NS - 1)
        def _():
            lo = (NS - 1) * RPT
            pltpu.sync_copy(y_hbm.at[pl.ds(lo, N - lo)],
                            acc_sh.at[pl.ds(lo, N - lo)])
            pltpu.sync_copy(zeros_hbm.at[pl.ds(N, NPD - N)],
                            acc_sh.at[pl.ds(N, NPD - N)])

    @pl.when(c == 1)
    def _():
        pltpu.sync_copy(zeros_hbm.at[pl.ds(s * RPT, RPT)],
                        acc_sh.at[pl.ds(s * RPT, RPT)])

    plsc.subcore_barrier()
    base = wid * NCHUNK

    def fire_idx(setn, cstart):
        for b in range(SB):
            pltpu.async_copy(edges_hbm.at[base + cstart + b], idx2.at[setn, b],
                             sem_i)

    def drain_idx(setn):
        for b in range(SB):
            pltpu.make_async_copy(edges_hbm.at[0], idx2.at[setn, b],
                                  sem_i).wait()

    def fire_gather(setn):
        for b in range(SB):
            pltpu.async_copy(y_hbm.at[idx2.at[setn, b, 0]], rows.at[setn, b],
                             sem_g)

    def drain_gather(setn):
        for b in range(SB):
            pltpu.make_async_copy(y_hbm.at[idx2.at[setn, b, 0]],
                                  rows.at[setn, b], sem_g).wait()

    def fire_scatter(setn):
        for b in range(SB):
            pltpu.async_copy(rows.at[setn, b], acc_sh.at[idx2.at[setn, b, 1]],
                             sem_s, add=True)

    def drain_scatter(setn):
        for b in range(SB):
            pltpu.make_async_copy(rows.at[setn, b], acc_sh.at[idx2.at[setn, b, 1]],
                                  sem_s).wait()

    fire_idx(0, 0)

    def body(k, carry):
        drain_idx(0)
        fire_gather(0)

        @pl.when(k > 0)
        def _():
            drain_scatter(1)

        fire_idx(1, k * 2 * SB + SB)
        drain_gather(0)
        fire_scatter(0)
        drain_idx(1)
        fire_gather(1)
        drain_scatter(0)

        @pl.when(k < NIT - 1)
        def _():
            fire_idx(0, (k + 1) * 2 * SB)

        drain_gather(1)
        fire_scatter(1)
        return carry

    lax.fori_loop(0, NIT, body, 0)
    drain_scatter(1)
    for t in range(PTAIL):
        pltpu.sync_copy(edges_hbm.at[base + NIT * 2 * SB + t], idx2.at[0, t])
        pltpu.async_copy(y_hbm.at[idx2.at[0, t, 0]], rows.at[0, t], sem_g).wait()
        pltpu.sync_copy(rows.at[0, t], acc_sh.at[idx2.at[0, t, 1]], add=True)
    plsc.subcore_barrier()
    pltpu.sync_copy(acc_sh.at[pl.ds(s * RPT, RPT)],
                    acc_out.at[c, pl.ds(s * RPT, RPT)])


_prop_kernel = functools.partial(
    pl.kernel,
    out_type=jax.ShapeDtypeStruct((NC, NPD, D), jnp.float32),
    mesh=_mesh,
    scratch_types=[
        pltpu.VMEM((2, SB, 2, C), jnp.int32),
        pltpu.VMEM((2, SB, C, D), jnp.float32),
        pltpu.VMEM_SHARED((NPD, D), jnp.float32),
        pltpu.SemaphoreType.DMA,
        pltpu.SemaphoreType.DMA,
        pltpu.SemaphoreType.DMA,
    ],
)(_prop_body)


_ROWS_B = 2000


def _dinv_scale_body(degp_ref, x_ref, dinv_ref, y_ref):
    di = lax.rsqrt(degp_ref[0] + degp_ref[1] + 1.0)
    dinv_ref[...] = di
    y_ref[...] = x_ref[...] * di


def _dinv_scale_call(degp, x):
    return pl.pallas_call(
        _dinv_scale_body,
        grid=(N // _ROWS_B,),
        in_specs=[
            pl.BlockSpec((NC, _ROWS_B, 1), lambda i: (0, i, 0)),
            pl.BlockSpec((_ROWS_B, D), lambda i: (i, 0)),
        ],
        out_specs=[
            pl.BlockSpec((_ROWS_B, 1), lambda i: (i, 0)),
            pl.BlockSpec((_ROWS_B, D), lambda i: (i, 0)),
        ],
        out_shape=[
            jax.ShapeDtypeStruct((N, 1), jnp.float32),
            jax.ShapeDtypeStruct((N, D), jnp.float32),
        ],
    )(degp, x)


def _dense_body(relu_scale, accp_ref, dinv_ref, w_ref, b_ref, out_ref):
    di = dinv_ref[...]
    z = di * (accp_ref[0] + accp_ref[1])
    h = jnp.dot(z, w_ref[...], preferred_element_type=jnp.float32,
                precision=lax.Precision.HIGHEST) + b_ref[...]
    if relu_scale:
        h = di * jnp.maximum(h, 0.0)
    out_ref[...] = h


def _dense_call(accp, dinv_col, w, b, relu_scale):
    return pl.pallas_call(
        functools.partial(_dense_body, relu_scale),
        grid=(N // _ROWS_B,),
        in_specs=[
            pl.BlockSpec((NC, _ROWS_B, D), lambda i: (0, i, 0)),
            pl.BlockSpec((_ROWS_B, 1), lambda i: (i, 0)),
            pl.BlockSpec((D, D), lambda i: (0, 0)),
            pl.BlockSpec((1, D), lambda i: (0, 0)),
        ],
        out_specs=pl.BlockSpec((_ROWS_B, D), lambda i: (i, 0)),
        out_shape=jax.ShapeDtypeStruct((N, D), jnp.float32),
    )(accp, dinv_col, w, b)


def kernel(edge_index, node_emb, W1, b1, W2, b2):
    edges = jnp.stack([edge_index[0].reshape(NCHT, C),
                       edge_index[1].reshape(NCHT, C)], axis=1)
    zeros_nd = jnp.zeros((NPD, D), jnp.float32)
    zeros_n = jnp.zeros((NPD,), jnp.float32)
    ones_c = jnp.ones((C,), jnp.float32)

    degp = _deg_kernel(edges, zeros_n, ones_c).reshape(NC, NPD, 1)
    dinv_col, y0 = _dinv_scale_call(degp, node_emb)

    acc0 = _prop_kernel(edges, y0, zeros_nd)
    y1 = _dense_call(acc0, dinv_col, W1, b1.reshape(1, D), relu_scale=True)
    acc1 = _prop_kernel(edges, y1, zeros_nd)
    out = _dense_call(acc1, dinv_col, W2, b2.reshape(1, D), relu_scale=False)
    return out


# speedup vs baseline: 1.0382x; 1.0382x over previous
import functools

import jax
import jax.numpy as jnp
from jax import lax
from jax.experimental import pallas as pl
from jax.experimental.pallas import tpu as pltpu
from jax.experimental.pallas import tpu_sc as plsc

N = 10000
E = 320000
D = 128

NC = 2
NS = 16
NW = NC * NS
EPW = E // NW
C = 80
NCHUNK = EPW // C
NCHT = E // C
NBUF = 5
NGRP = NCHUNK // NBUF
SB = 2
NIT = 31
PTAIL = NCHUNK - NIT * 2 * SB
NPD = 10240
RPT = NPD // NS
YRT = N // NS

_mesh = plsc.VectorSubcoreMesh(core_axis_name="c", subcore_axis_name="s",
                               num_cores=NC, num_subcores=NS)


def _deg_body(edges_hbm, zeros_hbm, ones_hbm, deg_out, dst_v, ones_v, acc_sh,
              sem_i, sem_s):
    c = lax.axis_index("c")
    s = lax.axis_index("s")
    wid = s * NC + c
    pltpu.sync_copy(zeros_hbm.at[pl.ds(s * RPT, RPT)], acc_sh.at[pl.ds(s * RPT, RPT)])
    pltpu.sync_copy(ones_hbm, ones_v)
    plsc.subcore_barrier()
    base = wid * NCHUNK

    for b in range(NBUF):
        pltpu.async_copy(edges_hbm.at[base + b, 1], dst_v.at[b], sem_i)

    def body(g, carry):
        for b in range(NBUF):
            pltpu.make_async_copy(edges_hbm.at[0, 1], dst_v.at[b], sem_i).wait()
        descs = [pltpu.async_copy(ones_v, acc_sh.at[dst_v.at[b]], sem_s, add=True)
                 for b in range(NBUF)]
        for d in descs:
            d.wait()

        @pl.when(g < NGRP - 1)
        def _():
            for b in range(NBUF):
                pltpu.async_copy(edges_hbm.at[base + (g + 1) * NBUF + b, 1],
                                 dst_v.at[b], sem_i)
        return carry

    lax.fori_loop(0, NGRP, body, 0)
    plsc.subcore_barrier()
    pltpu.sync_copy(acc_sh.at[pl.ds(s * RPT, RPT)],
                    deg_out.at[c, pl.ds(s * RPT, RPT)])


_deg_kernel = functools.partial(
    pl.kernel,
    out_type=jax.ShapeDtypeStruct((NC, NPD), jnp.float32),
    mesh=_mesh,
    scratch_types=[
        pltpu.VMEM((NBUF, C), jnp.int32),
        pltpu.VMEM((C,), jnp.float32),
        pltpu.VMEM_SHARED((NPD,), jnp.float32),
        pltpu.SemaphoreType.DMA,
        pltpu.SemaphoreType.DMA,
    ],
)(_deg_body)


def _prop_body(edges_hbm, y_hbm, zeros_hbm, acc_out,
               idx2, rows, acc_sh, sem_i, sem_g, sem_s):
    c = lax.axis_index("c")
    s = lax.axis_index("s")
    wid = s * NC + c

    @pl.when(c == 0)
    def _():
        @pl.when(s < NS - 1)
        def _():
            pltpu.sync_copy(y_hbm.at[pl.ds(s * RPT, RPT)],
                            acc_sh.at[pl.ds(s * RPT, RPT)])

        @pl.when(s == NS - 1)
        def _():
            lo = (NS - 1) * RPT
            pltpu.sync_copy(y_hbm.at[pl.ds(lo, N - lo)],
                            acc_sh.at[pl.ds(lo, N - lo)])
            pltpu.sync_copy(zeros_hbm.at[pl.ds(N, NPD - N)],
                            acc_sh.at[pl.ds(N, NPD - N)])

    @pl.when(c == 1)
    def _():
        pltpu.sync_copy(zeros_hbm.at[pl.ds(s * RPT, RPT)],
                        acc_sh.at[pl.ds(s * RPT, RPT)])

    plsc.subcore_barrier()
    base = wid * NCHUNK

    def fire_idx(setn, cstart):
        for b in range(SB):
            pltpu.async_copy(edges_hbm.at[base + cstart + b], idx2.at[setn, b],
                             sem_i)

    def drain_idx(setn):
        for b in range(SB):
            pltpu.make_async_copy(edges_hbm.at[0], idx2.at[setn, b],
                                  sem_i).wait()

    def fire_gather(setn):
        for b in range(SB):
            pltpu.async_copy(y_hbm.at[idx2.at[setn, b, 0]], rows.at[setn, b],
                             sem_g)

    def drain_gather(setn):
        for b in range(SB):
            pltpu.make_async_copy(y_hbm.at[idx2.at[setn, b, 0]],
                                  rows.at[setn, b], sem_g).wait()

    def fire_scatter(setn):
        for b in range(SB):
            pltpu.async_copy(rows.at[setn, b], acc_sh.at[idx2.at[setn, b, 1]],
                             sem_s, add=True)

    def drain_scatter(setn):
        for b in range(SB):
            pltpu.make_async_copy(rows.at[setn, b], acc_sh.at[idx2.at[setn, b, 1]],
                                  sem_s).wait()

    fire_idx(0, 0)

    def body(k, carry):
        drain_idx(0)
        fire_gather(0)

        @pl.when(k > 0)
        def _():
            drain_scatter(1)

        fire_idx(1, k * 2 * SB + SB)
        drain_gather(0)
        fire_scatter(0)
        drain_idx(1)
        fire_gather(1)
        drain_scatter(0)

        @pl.when(k < NIT - 1)
        def _():
            fire_idx(0, (k + 1) * 2 * SB)

        drain_gather(1)
        fire_scatter(1)
        return carry

    lax.fori_loop(0, NIT, body, 0)
    drain_scatter(1)
    for t in range(PTAIL):
        pltpu.sync_copy(edges_hbm.at[base + NIT * 2 * SB + t], idx2.at[0, t])
        pltpu.async_copy(y_hbm.at[idx2.at[0, t, 0]], rows.at[0, t], sem_g).wait()
        pltpu.sync_copy(rows.at[0, t], acc_sh.at[idx2.at[0, t, 1]], add=True)
    plsc.subcore_barrier()
    pltpu.sync_copy(acc_sh.at[pl.ds(s * RPT, RPT)],
                    acc_out.at[c, pl.ds(s * RPT, RPT)])


_prop_kernel = functools.partial(
    pl.kernel,
    out_type=jax.ShapeDtypeStruct((NC, NPD, D), jnp.float32),
    mesh=_mesh,
    scratch_types=[
        pltpu.VMEM((2, SB, 2, C), jnp.int32),
        pltpu.VMEM((2, SB, C, D), jnp.float32),
        pltpu.VMEM_SHARED((NPD, D), jnp.float32),
        pltpu.SemaphoreType.DMA,
        pltpu.SemaphoreType.DMA,
        pltpu.SemaphoreType.DMA,
    ],
)(_prop_body)


_ROWS_B = 2000


def _dinv_scale_body(degp_ref, x_ref, dinv_ref, y_ref):
    di = lax.rsqrt(degp_ref[0] + degp_ref[1] + 1.0)
    dinv_ref[...] = di
    y_ref[...] = x_ref[...] * di


def _dinv_scale_call(degp, x):
    return pl.pallas_call(
        _dinv_scale_body,
        grid=(N // _ROWS_B,),
        in_specs=[
            pl.BlockSpec((NC, _ROWS_B, 1), lambda i: (0, i, 0)),
            pl.BlockSpec((_ROWS_B, D), lambda i: (i, 0)),
        ],
        out_specs=[
            pl.BlockSpec((_ROWS_B, 1), lambda i: (i, 0)),
            pl.BlockSpec((_ROWS_B, D), lambda i: (i, 0)),
        ],
        out_shape=[
            jax.ShapeDtypeStruct((N, 1), jnp.float32),
            jax.ShapeDtypeStruct((N, D), jnp.float32),
        ],
    )(degp, x)


def _dense_body(relu_scale, accp_ref, dinv_ref, w_ref, b_ref, out_ref):
    di = dinv_ref[...]
    z = di * (accp_ref[0] + accp_ref[1])
    h = jnp.dot(z, w_ref[...], preferred_element_type=jnp.float32,
                precision=lax.Precision.HIGHEST) + b_ref[...]
    if relu_scale:
        h = di * jnp.maximum(h, 0.0)
    out_ref[...] = h


def _dense_call(accp, dinv_col, w, b, relu_scale):
    return pl.pallas_call(
        functools.partial(_dense_body, relu_scale),
        grid=(N // _ROWS_B,),
        in_specs=[
            pl.BlockSpec((NC, _ROWS_B, D), lambda i: (0, i, 0)),
            pl.BlockSpec((_ROWS_B, 1), lambda i: (i, 0)),
            pl.BlockSpec((D, D), lambda i: (0, 0)),
            pl.BlockSpec((1, D), lambda i: (0, 0)),
        ],
        out_specs=pl.BlockSpec((_ROWS_B, D), lambda i: (i, 0)),
        out_shape=jax.ShapeDtypeStruct((N, D), jnp.float32),
    )(accp, dinv_col, w, b)


def kernel(edge_index, node_emb, W1, b1, W2, b2):
    edges = jnp.stack([edge_index[0].reshape(NCHT, C),
                       edge_index[1].reshape(NCHT, C)], axis=1)
    zeros_nd = jnp.zeros((NPD, D), jnp.float32)
    zeros_n = jnp.zeros((NPD,), jnp.float32)
    ones_c = jnp.ones((C,), jnp.float32)

    degp = _deg_kernel(edges, zeros_n, ones_c).reshape(NC, NPD, 1)
    dinv_col, y0 = _dinv_scale_call(degp, node_emb)

    acc0 = _prop_kernel(edges, y0, zeros_nd)
    y1 = _dense_call(acc0, dinv_col, W1, b1.reshape(1, D), relu_scale=True)
    acc1 = _prop_kernel(edges, y1, zeros_nd)
    out = _dense_call(acc1, dinv_col, W2, b2.reshape(1, D), relu_scale=False)
    return out


# speedup vs baseline: 1.0517x; 1.0130x over previous
import functools

import jax
import jax.numpy as jnp
from jax import lax
from jax.experimental import pallas as pl
from jax.experimental.pallas import tpu as pltpu
from jax.experimental.pallas import tpu_sc as plsc

N = 10000
E = 320000
D = 128

NC = 2
NS = 16
NW = NC * NS
EPW = E // NW
C = 80
NCHUNK = EPW // C
NBUF = 5
NGRP = NCHUNK // NBUF
SB = 2
NIT = 31
PTAIL = NCHUNK - NIT * 2 * SB
NPD = 10240
RPT = NPD // NS
DPT = NPD // NS

_mesh = plsc.VectorSubcoreMesh(core_axis_name="c", subcore_axis_name="s",
                               num_cores=NC, num_subcores=NS)


def _deg_body(dst_hbm, zeros_hbm, ones_hbm, deg_out, dst_v, ones_v, acc_sh,
              sem_i, sem_s):
    c = lax.axis_index("c")
    s = lax.axis_index("s")
    wid = s * NC + c
    pltpu.sync_copy(zeros_hbm.at[pl.ds(s * DPT, DPT)], acc_sh.at[pl.ds(s * DPT, DPT)])
    pltpu.sync_copy(ones_hbm, ones_v)
    plsc.subcore_barrier()
    base = wid * EPW

    for b in range(NBUF):
        pltpu.async_copy(dst_hbm.at[pl.ds(base + b * C, C)], dst_v.at[b], sem_i)

    def body(g, carry):
        for b in range(NBUF):
            pltpu.make_async_copy(dst_hbm.at[pl.ds(0, C)], dst_v.at[b], sem_i).wait()
        descs = [pltpu.async_copy(ones_v, acc_sh.at[dst_v.at[b]], sem_s, add=True)
                 for b in range(NBUF)]
        for d in descs:
            d.wait()

        @pl.when(g < NGRP - 1)
        def _():
            for b in range(NBUF):
                off = base + ((g + 1) * NBUF + b) * C
                pltpu.async_copy(dst_hbm.at[pl.ds(off, C)], dst_v.at[b], sem_i)
        return carry

    lax.fori_loop(0, NGRP, body, 0)
    plsc.subcore_barrier()
    pltpu.sync_copy(acc_sh.at[pl.ds(s * DPT, DPT)],
                    deg_out.at[c, pl.ds(s * DPT, DPT)])


_deg_kernel = functools.partial(
    pl.kernel,
    out_type=jax.ShapeDtypeStruct((NC, NPD), jnp.float32),
    mesh=_mesh,
    scratch_types=[
        pltpu.VMEM((NBUF, C), jnp.int32),
        pltpu.VMEM((C,), jnp.float32),
        pltpu.VMEM_SHARED((NPD,), jnp.float32),
        pltpu.SemaphoreType.DMA,
        pltpu.SemaphoreType.DMA,
    ],
)(_deg_body)


def _prop_body(src_hbm, dst_hbm, y_hbm, zeros_hbm, acc_out,
               idx_s, idx_d, rows, acc_sh, sem_i, sem_g, sem_s):
    c = lax.axis_index("c")
    s = lax.axis_index("s")
    wid = s * NC + c
    pltpu.sync_copy(zeros_hbm.at[pl.ds(s * RPT, RPT)], acc_sh.at[pl.ds(s * RPT, RPT)])
    plsc.subcore_barrier()
    base = wid * EPW

    def fire_idx(setn, cstart):
        for b in range(SB):
            off = base + (cstart + b) * C
            pltpu.async_copy(src_hbm.at[pl.ds(off, C)], idx_s.at[setn, b], sem_i)
            pltpu.async_copy(dst_hbm.at[pl.ds(off, C)], idx_d.at[setn, b], sem_i)

    def drain_idx(setn):
        for b in range(SB):
            pltpu.make_async_copy(src_hbm.at[pl.ds(0, C)], idx_s.at[setn, b],
                                  sem_i).wait()
            pltpu.make_async_copy(dst_hbm.at[pl.ds(0, C)], idx_d.at[setn, b],
                                  sem_i).wait()

    def fire_gather(setn):
        for b in range(SB):
            pltpu.async_copy(y_hbm.at[idx_s.at[setn, b]], rows.at[setn, b], sem_g)

    def drain_gather(setn):
        for b in range(SB):
            pltpu.make_async_copy(y_hbm.at[idx_s.at[setn, b]], rows.at[setn, b],
                                  sem_g).wait()

    def fire_scatter(setn):
        for b in range(SB):
            pltpu.async_copy(rows.at[setn, b], acc_sh.at[idx_d.at[setn, b]],
                             sem_s, add=True)

    def drain_scatter(setn):
        for b in range(SB):
            pltpu.make_async_copy(rows.at[setn, b], acc_sh.at[idx_d.at[setn, b]],
                                  sem_s).wait()

    fire_idx(0, 0)

    def body(k, carry):
        drain_idx(0)
        fire_gather(0)

        @pl.when(k > 0)
        def _():
            drain_scatter(1)

        fire_idx(1, k * 2 * SB + SB)
        drain_gather(0)
        fire_scatter(0)
        drain_idx(1)
        fire_gather(1)
        drain_scatter(0)

        @pl.when(k < NIT - 1)
        def _():
            fire_idx(0, (k + 1) * 2 * SB)

        drain_gather(1)
        fire_scatter(1)
        return carry

    lax.fori_loop(0, NIT, body, 0)
    drain_scatter(1)
    for t in range(PTAIL):
        off = base + (NIT * 2 * SB + t) * C
        pltpu.sync_copy(src_hbm.at[pl.ds(off, C)], idx_s.at[0, t])
        pltpu.sync_copy(dst_hbm.at[pl.ds(off, C)], idx_d.at[0, t])
        pltpu.async_copy(y_hbm.at[idx_s.at[0, t]], rows.at[0, t], sem_g).wait()
        pltpu.sync_copy(rows.at[0, t], acc_sh.at[idx_d.at[0, t]], add=True)
    plsc.subcore_barrier()
    pltpu.sync_copy(acc_sh.at[pl.ds(s * RPT, RPT)],
                    acc_out.at[c, pl.ds(s * RPT, RPT)])


_prop_kernel = functools.partial(
    pl.kernel,
    out_type=jax.ShapeDtypeStruct((NC, NPD, D), jnp.float32),
    mesh=_mesh,
    scratch_types=[
        pltpu.VMEM((2, SB, C), jnp.int32),
        pltpu.VMEM((2, SB, C), jnp.int32),
        pltpu.VMEM((2, SB, C, D), jnp.float32),
        pltpu.VMEM_SHARED((NPD, D), jnp.float32),
        pltpu.SemaphoreType.DMA,
        pltpu.SemaphoreType.DMA,
        pltpu.SemaphoreType.DMA,
    ],
)(_prop_body)


def _dinv_body(degp_ref, dinv_ref):
    d = degp_ref[0:1, :] + degp_ref[1:2, :] + 1.0
    dinv_ref[...] = lax.rsqrt(d)


def _dinv_call(degp):
    return pl.pallas_call(
        _dinv_body,
        out_shape=jax.ShapeDtypeStruct((1, NPD), jnp.float32),
    )(degp)


def _scale_body(x_ref, dinv_ref, y_ref):
    y_ref[...] = x_ref[...] * dinv_ref[...]


_ROWS_B = 2000


def _scale_call(x, dinv_col):
    return pl.pallas_call(
        _scale_body,
        grid=(N // _ROWS_B,),
        in_specs=[
            pl.BlockSpec((_ROWS_B, D), lambda i: (i, 0)),
            pl.BlockSpec((_ROWS_B, 1), lambda i: (i, 0)),
        ],
        out_specs=pl.BlockSpec((_ROWS_B, D), lambda i: (i, 0)),
        out_shape=jax.ShapeDtypeStruct((N, D), jnp.float32),
    )(x, dinv_col)


def _dense_body(relu, want_y, accp_ref, x_ref, dinv_ref, w_ref, b_ref, *outs):
    a = accp_ref[0] + accp_ref[1]
    di = dinv_ref[...]
    z = di * a + (di * di) * x_ref[...]
    h = jnp.dot(z, w_ref[...], preferred_element_type=jnp.float32,
                precision=lax.Precision.HIGHEST) + b_ref[...]
    if relu:
        h = jnp.maximum(h, 0.0)
    outs[0][...] = h
    if want_y:
        outs[1][...] = di * h


def _dense_call(accp, x, dinv_col, w, b, relu, want_y):
    nouts = 2 if want_y else 1
    out_shape = [jax.ShapeDtypeStruct((N, D), jnp.float32)] * nouts
    out_specs = [pl.BlockSpec((_ROWS_B, D), lambda i: (i, 0))] * nouts
    res = pl.pallas_call(
        functools.partial(_dense_body, relu, want_y),
        grid=(N // _ROWS_B,),
        in_specs=[
            pl.BlockSpec((NC, _ROWS_B, D), lambda i: (0, i, 0)),
            pl.BlockSpec((_ROWS_B, D), lambda i: (i, 0)),
            pl.BlockSpec((_ROWS_B, 1), lambda i: (i, 0)),
            pl.BlockSpec((D, D), lambda i: (0, 0)),
            pl.BlockSpec((1, D), lambda i: (0, 0)),
        ],
        out_specs=out_specs,
        out_shape=out_shape,
    )(accp, x, dinv_col, w, b)
    return res if want_y else (res[0],)


def kernel(edge_index, node_emb, W1, b1, W2, b2):
    src = edge_index[0]
    dst = edge_index[1]
    zeros_nd = jnp.zeros((NPD, D), jnp.float32)
    zeros_n = jnp.zeros((NPD,), jnp.float32)
    ones_c = jnp.ones((C,), jnp.float32)

    degp = _deg_kernel(dst, zeros_n, ones_c)
    dinv_row = _dinv_call(degp)
    dinv_col = dinv_row[0, :N].reshape(N, 1)

    y0 = _scale_call(node_emb, dinv_col)
    acc0 = _prop_kernel(src, dst, y0, zeros_nd)
    x1, y1 = _dense_call(acc0, node_emb, dinv_col, W1, b1.reshape(1, D),
                         relu=True, want_y=True)
    acc1 = _prop_kernel(src, dst, y1, zeros_nd)
    (out,) = _dense_call(acc1, x1, dinv_col, W2, b2.reshape(1, D),
                         relu=False, want_y=False)
    return out
